# baseline (speedup 1.0000x reference)
import jax
import jax.numpy as jnp
from jax import lax
from jax.experimental import pallas as pl
from jax.experimental.pallas import tpu as pltpu
from jax.experimental.pallas import tpu_sc as plsc

N = 10000
E = 160000
D = 256
HALF = 128

NUM_CORES = 2
NUM_SUBCORES = 16
CHUNK = 64
SB = 8
NSB = 21
NBUF = 4
EDGES_PER_TILE = E // NUM_SUBCORES
CHUNKS_PER_TILE = NSB * SB
EDGES_PAD = CHUNKS_PER_TILE * CHUNK
DUMMY_ROW = N
DUMMY_SPAN = 112
AGG_ROWS = 10112
ROWS_PER_TILE = AGG_ROWS // NUM_SUBCORES


def _sc_body(tab_w, tab_c, srcw, dstw, srcc, dstc, zeros_h, zeros16_h, ones_h,
             aggw, cntw, aggc, cntc,
             agg_sh, cnt_sh, sidx0, sidx1, sidx2, sdst0, sdst1, sdst2,
             gbuf0, gbuf1, gbuf2, gbuf3, ones_v,
             gsem0, gsem1, gsem2, gsem3, ssem0, ssem1, ssem2, ssem3,
             isem0, isem1, isem2, csem):
    h = lax.axis_index("c")
    s = lax.axis_index("s")

    sidx = (sidx0, sidx1, sidx2)
    sdst = (sdst0, sdst1, sdst2)
    gbuf = (gbuf0, gbuf1, gbuf2, gbuf3)
    gsem = (gsem0, gsem1, gsem2, gsem3)
    ssem = (ssem0, ssem1, ssem2, ssem3)
    isem = (isem0, isem1, isem2)

    pltpu.sync_copy(ones_h, ones_v)

    def zero_my_rows():
        base = s * ROWS_PER_TILE
        for k in range(4):
            pltpu.sync_copy(zeros_h, agg_sh.at[pl.ds(base + 128 * k, 128)])
            pltpu.sync_copy(zeros16_h,
                            cnt_sh.at[pl.ds(base + 128 * k, 128)])
        rem = ROWS_PER_TILE - 512
        pltpu.sync_copy(zeros_h.at[pl.ds(0, rem)],
                        agg_sh.at[pl.ds(base + 512, rem)])
        pltpu.sync_copy(zeros16_h.at[pl.ds(0, rem)],
                        cnt_sh.at[pl.ds(base + 512, rem)])

    def run_relation(tab, src_h, dst_h, cnt_core):
        duty = h == cnt_core

        def load_sb(sc, p):
            pltpu.async_copy(src_h.at[s, pl.ds(SB * sc, SB)], sidx[p],
                             isem[p])
            pltpu.async_copy(dst_h.at[s, pl.ds(SB * sc, SB)], sdst[p],
                             isem[p])

        def wait_sb(sc, p):
            pltpu.make_async_copy(src_h.at[s, pl.ds(SB * sc, SB)], sidx[p],
                                  isem[p]).wait()
            pltpu.make_async_copy(dst_h.at[s, pl.ds(SB * sc, SB)], sdst[p],
                                  isem[p]).wait()

        def xform(p):
            ref = sidx[p]

            def row(r, _):
                for j in range(CHUNK // 16):
                    v = ref[r, pl.ds(16 * j, 16)]
                    ref[r, pl.ds(16 * j, 16)] = v + v + h
                return 0
            lax.fori_loop(0, SB, row, 0)

        def wait_scatter(b):
            pltpu.make_async_copy(gbuf[b], agg_sh.at[sdst[0].at[0]],
                                  ssem[b]).wait()

        def chunk_body(sc, q, j):
            b = j % NBUF
            b2 = (j + 2) % NBUF
            pltpu.make_async_copy(tab.at[sidx[q].at[j]], gbuf[b],
                                  gsem[b]).wait()
            pltpu.async_copy(gbuf[b], agg_sh.at[sdst[q].at[j]], ssem[b],
                             add=True)

            @pl.when(duty)
            def _():
                pltpu.async_copy(ones_v, cnt_sh.at[sdst[q].at[j]], csem,
                                 add=True)

            if j < SB - 2:
                if q == 0 and j < 2:
                    @pl.when(sc > 0)
                    def _():
                        wait_scatter(b2)
                else:
                    wait_scatter(b2)
                pltpu.async_copy(tab.at[sidx[q].at[j + 2]], gbuf[b2],
                                 gsem[b2])
            else:
                @pl.when(sc + 1 < NSB)
                def _():
                    wait_scatter(b2)
                    pltpu.async_copy(tab.at[sidx[(q + 1) % 3].at[j - 6]],
                                     gbuf[b2], gsem[b2])

        load_sb(0, 0)
        wait_sb(0, 0)
        xform(0)
        load_sb(1, 1)
        pltpu.async_copy(tab.at[sidx0.at[0]], gbuf0, gsem0)
        pltpu.async_copy(tab.at[sidx0.at[1]], gbuf1, gsem1)

        def tri(k, _):
            for q in range(3):
                sc = 3 * k + q

                @pl.when(sc + 1 < NSB)
                def _():
                    wait_sb(sc + 1, (q + 1) % 3)
                    xform((q + 1) % 3)

                for j in range(SB):
                    chunk_body(sc, q, j)

                @pl.when(duty)
                def _():
                    def drain(i, _):
                        pltpu.make_async_copy(ones_v,
                                              cnt_sh.at[sdst[q].at[0]],
                                              csem).wait()
                        return 0
                    lax.fori_loop(0, SB, drain, 0)

                @pl.when(sc + 2 < NSB)
                def _():
                    load_sb(sc + 2, (q + 2) % 3)
            return 0
        lax.fori_loop(0, NSB // 3, tri, 0)

        for b in range(NBUF):
            wait_scatter(b)

    def flush(agg_out, cnt_out, cnt_core):
        base = s * ROWS_PER_TILE
        pltpu.sync_copy(agg_sh.at[pl.ds(base, ROWS_PER_TILE)],
                        agg_out.at[h, pl.ds(base, ROWS_PER_TILE)])

        @pl.when(h == cnt_core)
        def _():
            pltpu.sync_copy(cnt_sh.at[pl.ds(base, ROWS_PER_TILE)],
                            cnt_out.at[pl.ds(base, ROWS_PER_TILE)])

    zero_my_rows()
    plsc.subcore_barrier()
    run_relation(tab_w, srcw, dstw, 0)
    plsc.subcore_barrier()
    flush(aggw, cntw, 0)
    zero_my_rows()
    plsc.subcore_barrier()
    run_relation(tab_c, srcc, dstc, 1)
    plsc.subcore_barrier()
    flush(aggc, cntc, 1)


def _sc_aggregate(tab_w, tab_c, srcw, dstw, srcc, dstc, zeros_h, zeros16_h,
                  ones_h):
    mesh = plsc.VectorSubcoreMesh(core_axis_name="c", subcore_axis_name="s")
    f32 = jnp.float32
    i32 = jnp.int32
    return pl.kernel(
        _sc_body,
        out_type=(
            jax.ShapeDtypeStruct((NUM_CORES, AGG_ROWS, HALF), f32),
            jax.ShapeDtypeStruct((AGG_ROWS, 16), f32),
            jax.ShapeDtypeStruct((NUM_CORES, AGG_ROWS, HALF), f32),
            jax.ShapeDtypeStruct((AGG_ROWS, 16), f32),
        ),
        mesh=mesh,
        compiler_params=pltpu.CompilerParams(use_tc_tiling_on_sc=False),
        scratch_types=[
            pltpu.VMEM_SHARED((AGG_ROWS, HALF), f32),
            pltpu.VMEM_SHARED((AGG_ROWS, 16), f32),
            pltpu.VMEM((SB, CHUNK), i32),
            pltpu.VMEM((SB, CHUNK), i32),
            pltpu.VMEM((SB, CHUNK), i32),
            pltpu.VMEM((SB, CHUNK), i32),
            pltpu.VMEM((SB, CHUNK), i32),
            pltpu.VMEM((SB, CHUNK), i32),
            pltpu.VMEM((CHUNK, HALF), f32),
            pltpu.VMEM((CHUNK, HALF), f32),
            pltpu.VMEM((CHUNK, HALF), f32),
            pltpu.VMEM((CHUNK, HALF), f32),
            pltpu.VMEM((CHUNK, 16), f32),
            pltpu.SemaphoreType.DMA,
            pltpu.SemaphoreType.DMA,
            pltpu.SemaphoreType.DMA,
            pltpu.SemaphoreType.DMA,
            pltpu.SemaphoreType.DMA,
            pltpu.SemaphoreType.DMA,
            pltpu.SemaphoreType.DMA,
            pltpu.SemaphoreType.DMA,
            pltpu.SemaphoreType.DMA,
            pltpu.SemaphoreType.DMA,
            pltpu.SemaphoreType.DMA,
            pltpu.SemaphoreType.DMA,
        ],
    )(tab_w, tab_c, srcw, dstw, srcc, dstc, zeros_h, zeros16_h, ones_h)


def _tc_body(x_ref, aggw_ref, cntw_ref, aggc_ref, cntc_ref,
             wnw_ref, wsw_ref, wnc_ref, wsc_ref, bw_ref, bc_ref, out_ref):
    f32 = jnp.float32
    ws = wsw_ref[...] + wsc_ref[...]
    acc = jnp.dot(x_ref[...], ws, preferred_element_type=f32)

    rw = 1.0 / jnp.maximum(cntw_ref[:, 0:1], 1.0)
    wnw = wnw_ref[...]
    acc += jnp.dot(aggw_ref[0] * rw, wnw[0:HALF, :], preferred_element_type=f32)
    acc += jnp.dot(aggw_ref[1] * rw, wnw[HALF:D, :], preferred_element_type=f32)

    rc = 1.0 / jnp.maximum(cntc_ref[:, 0:1], 1.0)
    wnc = wnc_ref[...]
    acc += jnp.dot(aggc_ref[0] * rc, wnc[0:HALF, :], preferred_element_type=f32)
    acc += jnp.dot(aggc_ref[1] * rc, wnc[HALF:D, :], preferred_element_type=f32)

    out_ref[...] = acc + bw_ref[...] + bc_ref[...]


def _tc_combine(x_paper, aggw, cntw, aggc, cntc, Wnw, Wsw, Wnc, Wsc, bw, bc):
    BLK = 1000
    grid = N // BLK
    full = lambda i: (0, 0)
    return pl.pallas_call(
        _tc_body,
        grid=(grid,),
        in_specs=[
            pl.BlockSpec((BLK, D), lambda i: (i, 0)),
            pl.BlockSpec((NUM_CORES, BLK, HALF), lambda i: (0, i, 0)),
            pl.BlockSpec((BLK, 16), lambda i: (i, 0)),
            pl.BlockSpec((NUM_CORES, BLK, HALF), lambda i: (0, i, 0)),
            pl.BlockSpec((BLK, 16), lambda i: (i, 0)),
            pl.BlockSpec((D, D), full),
            pl.BlockSpec((D, D), full),
            pl.BlockSpec((D, D), full),
            pl.BlockSpec((D, D), full),
            pl.BlockSpec((1, D), full),
            pl.BlockSpec((1, D), full),
        ],
        out_specs=pl.BlockSpec((BLK, D), lambda i: (i, 0)),
        out_shape=jax.ShapeDtypeStruct((N, D), jnp.float32),
    )(x_paper, aggw, cntw, aggc, cntc, Wnw, Wsw, Wnc, Wsc, bw, bc)


def _prep_idx(idx, is_dst):
    a = idx.reshape(NUM_SUBCORES, EDGES_PER_TILE)
    npad = EDGES_PAD - EDGES_PER_TILE
    if is_dst:
        pad = DUMMY_ROW + (jnp.arange(npad, dtype=jnp.int32) % DUMMY_SPAN)
        pad = jnp.broadcast_to(pad, (NUM_SUBCORES, npad))
    else:
        pad = jnp.zeros((NUM_SUBCORES, npad), jnp.int32)
    return jnp.concatenate([a, pad], axis=1).reshape(
        NUM_SUBCORES, CHUNKS_PER_TILE, CHUNK)


@jax.jit
def kernel(x_paper, x_author, edge_index_writes, edge_index_cites,
           W_neigh_writes, W_self_writes, b_writes,
           W_neigh_cites, W_self_cites, b_cites):
    tab_w = x_author.reshape(2 * N, HALF)
    tab_c = x_paper.reshape(2 * N, HALF)
    srcw = _prep_idx(edge_index_writes[0], False)
    dstw = _prep_idx(edge_index_writes[1], True)
    srcc = _prep_idx(edge_index_cites[0], False)
    dstc = _prep_idx(edge_index_cites[1], True)
    zeros_h = jnp.zeros((128, HALF), jnp.float32)
    zeros16_h = jnp.zeros((128, 16), jnp.float32)
    ones_h = jnp.ones((CHUNK, 16), jnp.float32)

    aggw, cntw, aggc, cntc = _sc_aggregate(
        tab_w, tab_c, srcw, dstw, srcc, dstc, zeros_h, zeros16_h, ones_h)

    return _tc_combine(x_paper, aggw, cntw, aggc, cntc,
                       W_neigh_writes, W_self_writes,
                       W_neigh_cites, W_self_cites,
                       b_writes.reshape(1, D), b_cites.reshape(1, D))


# speedup vs baseline: 2.0728x; 2.0728x over previous
import functools

import jax
import jax.numpy as jnp
from jax import lax
from jax.experimental import pallas as pl
from jax.experimental.pallas import tpu as pltpu
from jax.experimental.pallas import tpu_sc as plsc

N = 10000
E = 160000
D = 256
HALF = 128

NUM_CORES = 2
NUM_SUBCORES = 16
CHUNK = 64
SB = 8
NBUF = 4
EDGES_PER_TILE = E // NUM_SUBCORES
CHUNKS_PER_TILE = 160
NSB = CHUNKS_PER_TILE // SB
EDGES_PAD = CHUNKS_PER_TILE * CHUNK
DUMMY_ROW = N
DUMMY_SPAN = 112
AGG_ROWS = 10112
ROWS_PER_TILE = AGG_ROWS // NUM_SUBCORES


def _sc_body(tab_w, tab_c, srcw, dstw, srcc, dstc, zeros_h, zeros16_h, ones_h,
             aggw, cntw, aggc, cntc,
             agg_sh, cnt_sh, sidx0, sidx1, sdst0, sdst1,
             gbuf0, gbuf1, gbuf2, gbuf3, ones_v,
             gsem0, gsem1, gsem2, gsem3, isem0, isem1, csem):
    h = lax.axis_index("c")
    s = lax.axis_index("s")

    pltpu.sync_copy(ones_h, ones_v)

    def zero_my_rows():
        base = s * ROWS_PER_TILE
        for k in range(4):
            pltpu.sync_copy(zeros_h, agg_sh.at[pl.ds(base + 128 * k, 128)])
            pltpu.sync_copy(zeros16_h,
                            cnt_sh.at[pl.ds(base + 128 * k, 128)])
        rem = ROWS_PER_TILE - 512
        pltpu.sync_copy(zeros_h.at[pl.ds(0, rem)],
                        agg_sh.at[pl.ds(base + 512, rem)])
        pltpu.sync_copy(zeros16_h.at[pl.ds(0, rem)],
                        cnt_sh.at[pl.ds(base + 512, rem)])

    def run_relation(tab, src_h, dst_h, cnt_core):
        duty = h == cnt_core
        sidx = (sidx0, sidx1)
        sdst = (sdst0, sdst1)
        gbuf = (gbuf0, gbuf1, gbuf2, gbuf3)
        gsem = (gsem0, gsem1, gsem2, gsem3)
        isem = (isem0, isem1)

        def load_sb(sc, p):
            pltpu.async_copy(src_h.at[s, pl.ds(SB * sc, SB)], sidx[p], isem[p])
            pltpu.async_copy(dst_h.at[s, pl.ds(SB * sc, SB)], sdst[p], isem[p])

        def wait_sb(sc, p):
            pltpu.make_async_copy(src_h.at[s, pl.ds(SB * sc, SB)], sidx[p],
                                  isem[p]).wait()
            pltpu.make_async_copy(dst_h.at[s, pl.ds(SB * sc, SB)], sdst[p],
                                  isem[p]).wait()

        def xform(p):
            ref = sidx[p]

            def row(r, _):
                for j in range(CHUNK // 16):
                    v = ref[r, pl.ds(16 * j, 16)]
                    ref[r, pl.ds(16 * j, 16)] = v + v + h
                return 0
            lax.fori_loop(0, SB, row, 0)

        load_sb(0, 0)
        wait_sb(0, 0)
        xform(0)
        load_sb(1, 1)
        for b in range(NBUF):
            pltpu.async_copy(tab.at[sidx0.at[b]], gbuf[b], gsem[b])

        def pair(k, _):
            for p in range(2):
                sc = 2 * k + p

                @pl.when(sc + 1 < NSB)
                def _():
                    wait_sb(sc + 1, p ^ 1)
                    xform(p ^ 1)

                for j in range(SB):
                    b = j % NBUF
                    pltpu.make_async_copy(tab.at[sidx[p].at[j]], gbuf[b],
                                          gsem[b]).wait()
                    pltpu.sync_copy(gbuf[b], agg_sh.at[sdst[p].at[j]],
                                    add=True)

                    @pl.when(duty)
                    def _():
                        pltpu.async_copy(ones_v, cnt_sh.at[sdst[p].at[j]],
                                         csem, add=True)
                    if j < SB - NBUF:
                        pltpu.async_copy(tab.at[sidx[p].at[j + NBUF]],
                                         gbuf[b], gsem[b])
                    else:
                        @pl.when(sc + 1 < NSB)
                        def _():
                            pltpu.async_copy(
                                tab.at[sidx[p ^ 1].at[j - (SB - NBUF)]],
                                gbuf[b], gsem[b])

                @pl.when(duty)
                def _():
                    def drain(i, _):
                        pltpu.make_async_copy(ones_v,
                                              cnt_sh.at[sdst[p].at[0]],
                                              csem).wait()
                        return 0
                    lax.fori_loop(0, SB, drain, 0)

                @pl.when(sc + 2 < NSB)
                def _():
                    load_sb(sc + 2, p)
            return 0
        lax.fori_loop(0, NSB // 2, pair, 0)

    def flush(agg_out, cnt_out, cnt_core):
        base = s * ROWS_PER_TILE
        pltpu.sync_copy(agg_sh.at[pl.ds(base, ROWS_PER_TILE)],
                        agg_out.at[h, pl.ds(base, ROWS_PER_TILE)])

        @pl.when(h == cnt_core)
        def _():
            pltpu.sync_copy(cnt_sh.at[pl.ds(base, ROWS_PER_TILE)],
                            cnt_out.at[pl.ds(base, ROWS_PER_TILE)])

    zero_my_rows()
    plsc.subcore_barrier()
    run_relation(tab_w, srcw, dstw, 0)
    plsc.subcore_barrier()
    flush(aggw, cntw, 0)
    zero_my_rows()
    plsc.subcore_barrier()
    run_relation(tab_c, srcc, dstc, 1)
    plsc.subcore_barrier()
    flush(aggc, cntc, 1)


def _sc_aggregate(tab_w, tab_c, srcw, dstw, srcc, dstc, zeros_h, zeros16_h,
                  ones_h):
    mesh = plsc.VectorSubcoreMesh(core_axis_name="c", subcore_axis_name="s")
    f32 = jnp.float32
    return pl.kernel(
        _sc_body,
        out_type=(
            jax.ShapeDtypeStruct((NUM_CORES, AGG_ROWS, HALF), f32),
            jax.ShapeDtypeStruct((AGG_ROWS, 16), f32),
            jax.ShapeDtypeStruct((NUM_CORES, AGG_ROWS, HALF), f32),
            jax.ShapeDtypeStruct((AGG_ROWS, 16), f32),
        ),
        mesh=mesh,
        compiler_params=pltpu.CompilerParams(use_tc_tiling_on_sc=False),
        scratch_types=[
            pltpu.VMEM_SHARED((AGG_ROWS, HALF), f32),
            pltpu.VMEM_SHARED((AGG_ROWS, 16), f32),
            pltpu.VMEM((SB, CHUNK), jnp.int32),
            pltpu.VMEM((SB, CHUNK), jnp.int32),
            pltpu.VMEM((SB, CHUNK), jnp.int32),
            pltpu.VMEM((SB, CHUNK), jnp.int32),
            pltpu.VMEM((CHUNK, HALF), f32),
            pltpu.VMEM((CHUNK, HALF), f32),
            pltpu.VMEM((CHUNK, HALF), f32),
            pltpu.VMEM((CHUNK, HALF), f32),
            pltpu.VMEM((CHUNK, 16), f32),
            pltpu.SemaphoreType.DMA,
            pltpu.SemaphoreType.DMA,
            pltpu.SemaphoreType.DMA,
            pltpu.SemaphoreType.DMA,
            pltpu.SemaphoreType.DMA,
            pltpu.SemaphoreType.DMA,
            pltpu.SemaphoreType.DMA,
        ],
    )(tab_w, tab_c, srcw, dstw, srcc, dstc, zeros_h, zeros16_h, ones_h)


def _tc_body(x_ref, aggw_ref, cntw_ref, aggc_ref, cntc_ref,
             wnw_ref, wsw_ref, wnc_ref, wsc_ref, bw_ref, bc_ref, out_ref):
    f32 = jnp.float32
    ws = wsw_ref[...] + wsc_ref[...]
    acc = jnp.dot(x_ref[...], ws, preferred_element_type=f32)

    rw = 1.0 / jnp.maximum(cntw_ref[:, 0:1], 1.0)
    wnw = wnw_ref[...]
    acc += jnp.dot(aggw_ref[0] * rw, wnw[0:HALF, :], preferred_element_type=f32)
    acc += jnp.dot(aggw_ref[1] * rw, wnw[HALF:D, :], preferred_element_type=f32)

    rc = 1.0 / jnp.maximum(cntc_ref[:, 0:1], 1.0)
    wnc = wnc_ref[...]
    acc += jnp.dot(aggc_ref[0] * rc, wnc[0:HALF, :], preferred_element_type=f32)
    acc += jnp.dot(aggc_ref[1] * rc, wnc[HALF:D, :], preferred_element_type=f32)

    out_ref[...] = acc + bw_ref[...] + bc_ref[...]


def _tc_combine(x_paper, aggw, cntw, aggc, cntc, Wnw, Wsw, Wnc, Wsc, bw, bc):
    BLK = 1000
    grid = N // BLK
    full = lambda i: (0, 0)
    return pl.pallas_call(
        _tc_body,
        grid=(grid,),
        in_specs=[
            pl.BlockSpec((BLK, D), lambda i: (i, 0)),
            pl.BlockSpec((NUM_CORES, BLK, HALF), lambda i: (0, i, 0)),
            pl.BlockSpec((BLK, 16), lambda i: (i, 0)),
            pl.BlockSpec((NUM_CORES, BLK, HALF), lambda i: (0, i, 0)),
            pl.BlockSpec((BLK, 16), lambda i: (i, 0)),
            pl.BlockSpec((D, D), full),
            pl.BlockSpec((D, D), full),
            pl.BlockSpec((D, D), full),
            pl.BlockSpec((D, D), full),
            pl.BlockSpec((1, D), full),
            pl.BlockSpec((1, D), full),
        ],
        out_specs=pl.BlockSpec((BLK, D), lambda i: (i, 0)),
        out_shape=jax.ShapeDtypeStruct((N, D), jnp.float32),
    )(x_paper, aggw, cntw, aggc, cntc, Wnw, Wsw, Wnc, Wsc, bw, bc)


def _prep_idx(idx, is_dst):
    a = idx.reshape(NUM_SUBCORES, EDGES_PER_TILE)
    npad = EDGES_PAD - EDGES_PER_TILE
    if is_dst:
        pad = DUMMY_ROW + (jnp.arange(npad, dtype=jnp.int32) % DUMMY_SPAN)
        pad = jnp.broadcast_to(pad, (NUM_SUBCORES, npad))
    else:
        pad = jnp.zeros((NUM_SUBCORES, npad), jnp.int32)
    return jnp.concatenate([a, pad], axis=1).reshape(
        NUM_SUBCORES, CHUNKS_PER_TILE, CHUNK)


@jax.jit
def kernel(x_paper, x_author, edge_index_writes, edge_index_cites,
           W_neigh_writes, W_self_writes, b_writes,
           W_neigh_cites, W_self_cites, b_cites):
    tab_w = x_author.reshape(2 * N, HALF)
    tab_c = x_paper.reshape(2 * N, HALF)
    srcw = _prep_idx(edge_index_writes[0], False)
    dstw = _prep_idx(edge_index_writes[1], True)
    srcc = _prep_idx(edge_index_cites[0], False)
    dstc = _prep_idx(edge_index_cites[1], True)
    zeros_h = jnp.zeros((128, HALF), jnp.float32)
    zeros16_h = jnp.zeros((128, 16), jnp.float32)
    ones_h = jnp.ones((CHUNK, 16), jnp.float32)

    aggw, cntw, aggc, cntc = _sc_aggregate(
        tab_w, tab_c, srcw, dstw, srcc, dstc, zeros_h, zeros16_h, ones_h)

    return _tc_combine(x_paper, aggw, cntw, aggc, cntc,
                       W_neigh_writes, W_self_writes,
                       W_neigh_cites, W_self_cites,
                       b_writes.reshape(1, D), b_cites.reshape(1, D))


# speedup vs baseline: 2.1010x; 1.0136x over previous
import jax
import jax.numpy as jnp
from jax import lax
from jax.experimental import pallas as pl
from jax.experimental.pallas import tpu as pltpu
from jax.experimental.pallas import tpu_sc as plsc

N = 10000
E = 160000
D = 256
HALF = 128

NUM_CORES = 2
NUM_SUBCORES = 16
CHUNK = 128
SB = 8
EDGES_PER_TILE = E // NUM_SUBCORES
CHUNKS_PER_TILE = 80
NSB = CHUNKS_PER_TILE // SB
EDGES_PAD = CHUNKS_PER_TILE * CHUNK
DUMMY_ROW = N
AGG_ROWS = 10112
ROWS_PER_TILE = AGG_ROWS // NUM_SUBCORES


def _sc_body(tab_w, tab_c, srcw, dstw, srcc, dstc, zeros_h, zeros16_h, ones_h,
             aggw, cntw, aggc, cntc,
             agg_sh, cnt_sh, sidx0, sidx1, sdst0, sdst1, gbuf0, gbuf1,
             ones_v, gsem0, gsem1, isem0, isem1, csem):
    h = lax.axis_index("c")
    s = lax.axis_index("s")

    pltpu.sync_copy(ones_h, ones_v)

    def zero_my_rows():
        base = s * ROWS_PER_TILE
        for k in range(4):
            pltpu.sync_copy(zeros_h, agg_sh.at[pl.ds(base + 128 * k, 128)])
            pltpu.sync_copy(zeros16_h,
                            cnt_sh.at[pl.ds(base + 128 * k, 128)])
        rem = ROWS_PER_TILE - 512
        pltpu.sync_copy(zeros_h.at[pl.ds(0, rem)],
                        agg_sh.at[pl.ds(base + 512, rem)])
        pltpu.sync_copy(zeros16_h.at[pl.ds(0, rem)],
                        cnt_sh.at[pl.ds(base + 512, rem)])

    def run_relation(tab, src_h, dst_h, cnt_core):
        duty = h == cnt_core
        sidx = (sidx0, sidx1)
        sdst = (sdst0, sdst1)
        gbuf = (gbuf0, gbuf1)
        gsem = (gsem0, gsem1)
        isem = (isem0, isem1)

        def load_sb(sc, p):
            pltpu.async_copy(src_h.at[s, pl.ds(SB * sc, SB)], sidx[p], isem[p])
            pltpu.async_copy(dst_h.at[s, pl.ds(SB * sc, SB)], sdst[p], isem[p])

        def wait_sb(sc, p):
            pltpu.make_async_copy(src_h.at[s, pl.ds(SB * sc, SB)], sidx[p],
                                  isem[p]).wait()
            pltpu.make_async_copy(dst_h.at[s, pl.ds(SB * sc, SB)], sdst[p],
                                  isem[p]).wait()

        def xform(p):
            ref = sidx[p]

            def row(r, _):
                for j in range(SB):
                    v = ref[r, pl.ds(16 * j, 16)]
                    ref[r, pl.ds(16 * j, 16)] = v + v + h
                return 0
            lax.fori_loop(0, SB, row, 0)

        load_sb(0, 0)
        wait_sb(0, 0)
        xform(0)
        load_sb(1, 1)
        pltpu.async_copy(tab.at[sidx0.at[0]], gbuf0, gsem0)
        pltpu.async_copy(tab.at[sidx0.at[1]], gbuf1, gsem1)

        def pair(k, _):
            for p in range(2):
                sc = 2 * k + p

                @pl.when(sc + 1 < NSB)
                def _():
                    wait_sb(sc + 1, p ^ 1)
                    xform(p ^ 1)

                for j in range(SB):
                    b = j % 2
                    pltpu.make_async_copy(tab.at[sidx[p].at[j]], gbuf[b],
                                          gsem[b]).wait()
                    pltpu.sync_copy(gbuf[b], agg_sh.at[sdst[p].at[j]],
                                    add=True)

                    @pl.when(duty)
                    def _():
                        pltpu.async_copy(ones_v, cnt_sh.at[sdst[p].at[j]],
                                         csem, add=True)
                    if j < SB - 2:
                        pltpu.async_copy(tab.at[sidx[p].at[j + 2]], gbuf[b],
                                         gsem[b])
                    else:
                        @pl.when(sc + 1 < NSB)
                        def _():
                            pltpu.async_copy(tab.at[sidx[p ^ 1].at[j - 6]],
                                             gbuf[b], gsem[b])

                @pl.when(duty)
                def _():
                    def drain(i, _):
                        pltpu.make_async_copy(ones_v,
                                              cnt_sh.at[sdst[p].at[0]],
                                              csem).wait()
                        return 0
                    lax.fori_loop(0, SB, drain, 0)

                @pl.when(sc + 2 < NSB)
                def _():
                    load_sb(sc + 2, p)
            return 0
        lax.fori_loop(0, NSB // 2, pair, 0)

    def flush(agg_out, cnt_out, cnt_core):
        base = s * ROWS_PER_TILE
        pltpu.sync_copy(agg_sh.at[pl.ds(base, ROWS_PER_TILE)],
                        agg_out.at[h, pl.ds(base, ROWS_PER_TILE)])

        @pl.when(h == cnt_core)
        def _():
            pltpu.sync_copy(cnt_sh.at[pl.ds(base, ROWS_PER_TILE)],
                            cnt_out.at[pl.ds(base, ROWS_PER_TILE)])

    zero_my_rows()
    plsc.subcore_barrier()
    run_relation(tab_w, srcw, dstw, 0)
    plsc.subcore_barrier()
    flush(aggw, cntw, 0)
    zero_my_rows()
    plsc.subcore_barrier()
    run_relation(tab_c, srcc, dstc, 1)
    plsc.subcore_barrier()
    flush(aggc, cntc, 1)


def _sc_aggregate(tab_w, tab_c, srcw, dstw, srcc, dstc, zeros_h, zeros16_h,
                  ones_h):
    mesh = plsc.VectorSubcoreMesh(core_axis_name="c", subcore_axis_name="s")
    f32 = jnp.float32
    return pl.kernel(
        _sc_body,
        out_type=(
            jax.ShapeDtypeStruct((NUM_CORES, AGG_ROWS, HALF), f32),
            jax.ShapeDtypeStruct((AGG_ROWS, 16), f32),
            jax.ShapeDtypeStruct((NUM_CORES, AGG_ROWS, HALF), f32),
            jax.ShapeDtypeStruct((AGG_ROWS, 16), f32),
        ),
        mesh=mesh,
        compiler_params=pltpu.CompilerParams(use_tc_tiling_on_sc=False),
        scratch_types=[
            pltpu.VMEM_SHARED((AGG_ROWS, HALF), f32),
            pltpu.VMEM_SHARED((AGG_ROWS, 16), f32),
            pltpu.VMEM((SB, CHUNK), jnp.int32),
            pltpu.VMEM((SB, CHUNK), jnp.int32),
            pltpu.VMEM((SB, CHUNK), jnp.int32),
            pltpu.VMEM((SB, CHUNK), jnp.int32),
            pltpu.VMEM((CHUNK, HALF), f32),
            pltpu.VMEM((CHUNK, HALF), f32),
            pltpu.VMEM((CHUNK, 16), f32),
            pltpu.SemaphoreType.DMA,
            pltpu.SemaphoreType.DMA,
            pltpu.SemaphoreType.DMA,
            pltpu.SemaphoreType.DMA,
            pltpu.SemaphoreType.DMA,
        ],
    )(tab_w, tab_c, srcw, dstw, srcc, dstc, zeros_h, zeros16_h, ones_h)


def _tc_body(x_ref, aggw_ref, cntw_ref, aggc_ref, cntc_ref,
             wnw_ref, wsw_ref, wnc_ref, wsc_ref, bw_ref, bc_ref, out_ref):
    f32 = jnp.float32
    ws = wsw_ref[...] + wsc_ref[...]
    acc = jnp.dot(x_ref[...], ws, preferred_element_type=f32)

    rw = 1.0 / jnp.maximum(cntw_ref[:, 0:1], 1.0)
    wnw = wnw_ref[...]
    acc += jnp.dot(aggw_ref[0] * rw, wnw[0:HALF, :], preferred_element_type=f32)
    acc += jnp.dot(aggw_ref[1] * rw, wnw[HALF:D, :], preferred_element_type=f32)

    rc = 1.0 / jnp.maximum(cntc_ref[:, 0:1], 1.0)
    wnc = wnc_ref[...]
    acc += jnp.dot(aggc_ref[0] * rc, wnc[0:HALF, :], preferred_element_type=f32)
    acc += jnp.dot(aggc_ref[1] * rc, wnc[HALF:D, :], preferred_element_type=f32)

    out_ref[...] = acc + bw_ref[...] + bc_ref[...]


def _tc_combine(x_paper, aggw, cntw, aggc, cntc, Wnw, Wsw, Wnc, Wsc, bw, bc):
    BLK = 1000
    grid = N // BLK
    full = lambda i: (0, 0)
    return pl.pallas_call(
        _tc_body,
        grid=(grid,),
        in_specs=[
            pl.BlockSpec((BLK, D), lambda i: (i, 0)),
            pl.BlockSpec((NUM_CORES, BLK, HALF), lambda i: (0, i, 0)),
            pl.BlockSpec((BLK, 16), lambda i: (i, 0)),
            pl.BlockSpec((NUM_CORES, BLK, HALF), lambda i: (0, i, 0)),
            pl.BlockSpec((BLK, 16), lambda i: (i, 0)),
            pl.BlockSpec((D, D), full),
            pl.BlockSpec((D, D), full),
            pl.BlockSpec((D, D), full),
            pl.BlockSpec((D, D), full),
            pl.BlockSpec((1, D), full),
            pl.BlockSpec((1, D), full),
        ],
        out_specs=pl.BlockSpec((BLK, D), lambda i: (i, 0)),
        out_shape=jax.ShapeDtypeStruct((N, D), jnp.float32),
    )(x_paper, aggw, cntw, aggc, cntc, Wnw, Wsw, Wnc, Wsc, bw, bc)


def _prep_idx(idx, pad_val):
    a = idx.reshape(NUM_SUBCORES, EDGES_PER_TILE)
    pad = jnp.full((NUM_SUBCORES, EDGES_PAD - EDGES_PER_TILE), pad_val,
                   jnp.int32)
    return jnp.concatenate([a, pad], axis=1).reshape(
        NUM_SUBCORES, CHUNKS_PER_TILE, CHUNK)


@jax.jit
def kernel(x_paper, x_author, edge_index_writes, edge_index_cites,
           W_neigh_writes, W_self_writes, b_writes,
           W_neigh_cites, W_self_cites, b_cites):
    tab_w = x_author.reshape(2 * N, HALF)
    tab_c = x_paper.reshape(2 * N, HALF)
    srcw = _prep_idx(edge_index_writes[0], 0)
    dstw = _prep_idx(edge_index_writes[1], DUMMY_ROW)
    srcc = _prep_idx(edge_index_cites[0], 0)
    dstc = _prep_idx(edge_index_cites[1], DUMMY_ROW)
    zeros_h = jnp.zeros((128, HALF), jnp.float32)
    zeros16_h = jnp.zeros((128, 16), jnp.float32)
    ones_h = jnp.ones((CHUNK, 16), jnp.float32)

    aggw, cntw, aggc, cntc = _sc_aggregate(
        tab_w, tab_c, srcw, dstw, srcc, dstc, zeros_h, zeros16_h, ones_h)

    return _tc_combine(x_paper, aggw, cntw, aggc, cntc,
                       W_neigh_writes, W_self_writes,
                       W_neigh_cites, W_self_cites,
                       b_writes.reshape(1, D), b_cites.reshape(1, D))
